# baseline (speedup 1.0000x reference)
import functools

import jax
import jax.numpy as jnp
from jax import lax
from jax.experimental import pallas as pl
from jax.experimental.pallas import tpu as pltpu
from jax.experimental.pallas import tpu_sc as plsc

_N = 10000
_E = 320000
_D = 128
_H = 16
_OUT = 128
_HF = _H * _OUT

_NC = 2
_NS = 16
_NW = _NC * _NS
_EPW = _E // _NW

_BM = 400
_G1 = 40
_GB = 80
_G2 = 16


def _tc_dense_body(x_ref, w1t, w2t, p1, p2, lw1t, lb1, lw2t, lb2,
                   h1o, h2o, to, xso):
    xb = x_ref[...]
    h1 = jnp.dot(xb, w1t[...], preferred_element_type=jnp.float32)
    h1o[...] = h1
    a1 = jnp.dot(h1, p1[...], preferred_element_type=jnp.float32)
    h2 = jnp.dot(xb, w2t[...], preferred_element_type=jnp.float32)
    h2o[...] = h2
    a2 = jnp.dot(h2, p2[...], preferred_element_type=jnp.float32)
    to[...] = jnp.concatenate(
        [a1[:, :_H], a2[:, _H:], a1[:, _H:], a2[:, :_H],
         jnp.zeros((_BM, 64), jnp.float32)], axis=1)
    x1 = jnp.dot(xb, lw1t[...], preferred_element_type=jnp.float32) + lb1[...]
    x1 = jnp.where(x1 > 0, x1, jnp.exp(x1) - 1.0)
    xs = jnp.dot(x1, lw2t[...], preferred_element_type=jnp.float32) + lb2[...]
    xso[...] = jnp.where(xs > 0, xs, jnp.exp(xs) - 1.0)


def _tc_dense(x, w1t, w2t, p1, p2, lw1t, lb1, lw2t, lb2):
    nb = _N // _BM
    full = lambda shape: pl.BlockSpec(shape, lambda i: (0, 0))
    return pl.pallas_call(
        _tc_dense_body,
        grid=(nb,),
        in_specs=[
            pl.BlockSpec((_BM, _D), lambda i: (i, 0)),
            full((_D, _HF)), full((_D, _HF)),
            full((_HF, 2 * _H)), full((_HF, 2 * _H)),
            full((_D, 4 * _OUT)), full((1, 4 * _OUT)),
            full((4 * _OUT, _OUT)), full((1, _OUT)),
        ],
        out_specs=[
            pl.BlockSpec((_BM, _HF), lambda i: (i, 0)),
            pl.BlockSpec((_BM, _HF), lambda i: (i, 0)),
            pl.BlockSpec((_BM, _D), lambda i: (i, 0)),
            pl.BlockSpec((_BM, _OUT), lambda i: (i, 0)),
        ],
        out_shape=[
            jax.ShapeDtypeStruct((_N, _HF), jnp.float32),
            jax.ShapeDtypeStruct((_N, _HF), jnp.float32),
            jax.ShapeDtypeStruct((_N, _D), jnp.float32),
            jax.ShapeDtypeStruct((_N, _OUT), jnp.float32),
        ],
    )(x, w1t, w2t, p1, p2, lw1t, lb1, lw2t, lb2)


def _zero_rows128(buf, nrows_buf):
    def zrow(j, _):
        for t in range(8):
            buf[j, pl.ds(t * 16, 16)] = jnp.zeros((16,), jnp.float32)
        return 0
    lax.fori_loop(0, nrows_buf, zrow, 0)


def _zero_table128(buf, dst, s, nrows_buf):
    @pl.when(s < 10)
    def _():
        for t in range(1000 // nrows_buf):
            pltpu.sync_copy(buf, dst.at[pl.ds(s * 1000 + t * nrows_buf,
                                              nrows_buf)])
        rem = 1000 % nrows_buf
        if rem:
            pltpu.sync_copy(
                buf.at[pl.ds(0, rem)],
                dst.at[pl.ds(s * 1000 + 1000 - rem, rem)])


def _sc_edgea_body(e0h, e1h, th,
                   w1h, w2h, d1ph,
                   idx0, idx1, bta, btb, bw1, bw2, bw1p, d1s,
                   sem0, sem1):
    c = lax.axis_index("c")
    s = lax.axis_index("s")
    wid = s * _NC + c

    _zero_rows128(bw1p, _G1)
    _zero_table128(bw1p, d1s, s, _G1)
    plsc.subcore_barrier()

    def chunk(k, _):
        b = pl.multiple_of(wid * _EPW + k * _G1, 8)
        cpi = pltpu.async_copy(e0h.at[pl.ds(b, _G1)], idx0, sem0)
        cpj = pltpu.async_copy(e1h.at[pl.ds(b, _G1)], idx1, sem1)
        cpi.wait()
        cp0 = pltpu.async_copy(th.at[idx0], bta, sem0)
        cpj.wait()
        cp1 = pltpu.async_copy(th.at[idx1], btb, sem1)
        cp0.wait()
        cp1.wait()

        def edge(i, _):
            s1 = bta[i, pl.ds(0, _H)] + btb[i, pl.ds(2 * _H, _H)]
            s2 = btb[i, pl.ds(3 * _H, _H)] + bta[i, pl.ds(_H, _H)]
            w1 = jnp.exp(jnp.maximum(s1, 0.2 * s1))
            bw1[i] = w1
            bw1p[i, pl.ds(0, _H)] = w1
            bw2[i] = jnp.exp(jnp.maximum(s2, 0.2 * s2))
            return 0
        lax.fori_loop(0, _G1, edge, 0)

        pltpu.sync_copy(bw1, w1h.at[pl.ds(b, _G1)])
        pltpu.sync_copy(bw2, w2h.at[pl.ds(b, _G1)])
        pltpu.sync_copy(bw1p, d1s.at[idx1], add=True)
        return 0
    lax.fori_loop(0, _EPW // _G1, chunk, 0)

    plsc.subcore_barrier()

    @pl.when(s < 10)
    def _():
        rows = pl.ds(s * 1000, 1000)
        pltpu.sync_copy(d1s.at[rows], d1ph.at[c].at[rows])


def _sc_edgea(e0, e1, tbl):
    mesh = plsc.VectorSubcoreMesh(core_axis_name="c", subcore_axis_name="s")
    f = pl.kernel(
        _sc_edgea_body,
        out_type=(
            jax.ShapeDtypeStruct((_E, _H), jnp.float32),
            jax.ShapeDtypeStruct((_E, _H), jnp.float32),
            jax.ShapeDtypeStruct((_NC, _N, _D), jnp.float32),
        ),
        mesh=mesh,
        scratch_types=[
            pltpu.VMEM((_G1,), jnp.int32),
            pltpu.VMEM((_G1,), jnp.int32),
            pltpu.VMEM((_G1, _D), jnp.float32),
            pltpu.VMEM((_G1, _D), jnp.float32),
            pltpu.VMEM((_G1, _H), jnp.float32),
            pltpu.VMEM((_G1, _H), jnp.float32),
            pltpu.VMEM((_G1, _D), jnp.float32),
            pltpu.VMEM_SHARED((_N, _D), jnp.float32),
            pltpu.SemaphoreType.DMA,
            pltpu.SemaphoreType.DMA,
        ],
    )
    return f(e0, e1, tbl)


def _sc_edgeb_body(e0h, w2h, d2ph,
                   idx0, bw2, bw2p, d2s):
    c = lax.axis_index("c")
    s = lax.axis_index("s")
    wid = s * _NC + c

    _zero_rows128(bw2p, _GB)
    _zero_table128(bw2p, d2s, s, _GB)
    plsc.subcore_barrier()

    def chunk(k, _):
        b = pl.multiple_of(wid * _EPW + k * _GB, 8)
        pltpu.sync_copy(e0h.at[pl.ds(b, _GB)], idx0)
        pltpu.sync_copy(w2h.at[pl.ds(b, _GB)], bw2)

        def edge(i, _):
            bw2p[i, pl.ds(0, _H)] = bw2[i]
            return 0
        lax.fori_loop(0, _GB, edge, 0)
        pltpu.sync_copy(bw2p, d2s.at[idx0], add=True)
        return 0
    lax.fori_loop(0, _EPW // _GB, chunk, 0)

    plsc.subcore_barrier()

    @pl.when(s < 10)
    def _():
        rows = pl.ds(s * 1000, 1000)
        pltpu.sync_copy(d2s.at[rows], d2ph.at[c].at[rows])


def _sc_edgeb(e0, w2):
    mesh = plsc.VectorSubcoreMesh(core_axis_name="c", subcore_axis_name="s")
    f = pl.kernel(
        _sc_edgeb_body,
        out_type=jax.ShapeDtypeStruct((_NC, _N, _D), jnp.float32),
        mesh=mesh,
        scratch_types=[
            pltpu.VMEM((_GB,), jnp.int32),
            pltpu.VMEM((_GB, _H), jnp.float32),
            pltpu.VMEM((_GB, _D), jnp.float32),
            pltpu.VMEM_SHARED((_N, _D), jnp.float32),
        ],
    )
    return f(e0, w2)


def _tc_comb_body(d1p_ref, d2p_ref, dd1_ref, dd2_ref):
    z = jnp.zeros((_BM, _D - _H), jnp.float32)
    dd1_ref[...] = jnp.concatenate(
        [1.0 / (d1p_ref[0, :, :_H] + d1p_ref[1, :, :_H] + 1e-16), z], axis=1)
    dd2_ref[...] = jnp.concatenate(
        [1.0 / (d2p_ref[0, :, :_H] + d2p_ref[1, :, :_H] + 1e-16), z], axis=1)


def _tc_comb(d1p, d2p):
    nb = _N // _BM
    return pl.pallas_call(
        _tc_comb_body,
        grid=(nb,),
        in_specs=[
            pl.BlockSpec((_NC, _BM, _D), lambda i: (0, i, 0)),
            pl.BlockSpec((_NC, _BM, _D), lambda i: (0, i, 0)),
        ],
        out_specs=[
            pl.BlockSpec((_BM, _D), lambda i: (i, 0)),
            pl.BlockSpec((_BM, _D), lambda i: (i, 0)),
        ],
        out_shape=[
            jax.ShapeDtypeStruct((_N, _D), jnp.float32),
            jax.ShapeDtypeStruct((_N, _D), jnp.float32),
        ],
    )(d1p, d2p)


_SUP = 40
_SUB = 8
_NSB = _SUP // _SUB


def _sc_agg_body(srch, dsth, wh, ddh, hh,
                 acco,
                 idxs0, idxd0, idxs1, idxd1, bw, bdg, bh0, bh1, bmsg, accs,
                 semd, semh0, semh1, semi0, semj0, semi1, semj1, semw):
    c = lax.axis_index("c")
    s = lax.axis_index("s")
    wid = s * _NC + c
    base = wid * _EPW
    bhs = (bh0, bh1)
    semhs = (semh0, semh1)
    idxss = (idxs0, idxs1)
    idxds = (idxd0, idxd1)
    semis = (semi0, semi1)
    semjs = (semj0, semj1)

    for r in range(_SUP):
        for t in range(8):
            bmsg[r, pl.ds(t * 16, 16)] = jnp.zeros((16,), jnp.float32)

    @pl.when(s < 10)
    def _():
        def zcp(t, _):
            pltpu.sync_copy(bmsg, accs.at[pl.ds(s * 1000 + t * _SUP, _SUP)])
            return 0
        lax.fori_loop(0, 1000 // _SUP, zcp, 0)
    plsc.subcore_barrier()

    def _issue_idx(k, p):
        bn = pl.multiple_of(base + k * _SUP, 8)
        pltpu.async_copy(srch.at[pl.ds(bn, _SUP)], idxss[p], semis[p])
        pltpu.async_copy(dsth.at[pl.ds(bn, _SUP)], idxds[p], semjs[p])

    def _super(k, p, prefetch):
        b = pl.multiple_of(base + k * _SUP, 8)
        idxs = idxss[p]
        idxd = idxds[p]
        pltpu.make_async_copy(srch.at[pl.ds(b, _SUP)], idxs, semis[p]).wait()
        pltpu.make_async_copy(dsth.at[pl.ds(b, _SUP)], idxd, semjs[p]).wait()
        cpd = pltpu.async_copy(ddh.at[idxd], bdg, semd)
        cps = [None] * _NSB
        cps[0] = pltpu.async_copy(hh.at[idxs.at[pl.ds(0, _SUB)]], bh0, semh0)
        cpw = pltpu.async_copy(wh.at[pl.ds(b, _SUP)], bw, semw)
        prefetch()
        cpw.wait()
        cpd.wait()

        for t in range(_NSB):
            if t + 1 < _NSB:
                cps[t + 1] = pltpu.async_copy(
                    hh.at[idxs.at[pl.ds((t + 1) * _SUB, _SUB)]],
                    bhs[(t + 1) % 2], semhs[(t + 1) % 2])
            cps[t].wait()
            bh = bhs[t % 2]

            def edge(i, _):
                e = t * _SUB + i
                al = bw[e] * bdg[e, pl.ds(0, _H)]
                mv = [jnp.zeros((16,), jnp.float32) for _ in range(8)]
                for hd in range(_H):
                    av = jnp.full((16,), al[hd], jnp.float32)
                    for j in range(8):
                        mv[j] = mv[j] + av * bh[i, pl.ds(hd * _OUT + j * 16,
                                                         16)]
                for j in range(8):
                    bmsg[e, pl.ds(j * 16, 16)] = mv[j]
                return 0
            lax.fori_loop(0, _SUB, edge, 0)

        pltpu.sync_copy(bmsg, accs.at[idxd], add=True)

    nsup = _EPW // _SUP
    _issue_idx(0, 0)

    def chunk2(k2, _):
        k = 2 * k2
        _super(k, 0, lambda: _issue_idx(k + 1, 1))

        def pf2():
            @pl.when(k2 < nsup // 2 - 1)
            def _():
                _issue_idx(k + 2, 0)
        _super(k + 1, 1, pf2)
        return 0
    lax.fori_loop(0, nsup // 2, chunk2, 0)

    plsc.subcore_barrier()

    @pl.when(s < 10)
    def _():
        rows = pl.ds(s * 1000, 1000)
        pltpu.sync_copy(accs.at[rows], acco.at[c].at[rows])


def _sc_agg(src, dst, w, dd, h):
    mesh = plsc.VectorSubcoreMesh(core_axis_name="c", subcore_axis_name="s")
    f = pl.kernel(
        _sc_agg_body,
        out_type=jax.ShapeDtypeStruct((_NC, _N, _OUT), jnp.float32),
        mesh=mesh,
        scratch_types=[
            pltpu.VMEM((_SUP,), jnp.int32),
            pltpu.VMEM((_SUP,), jnp.int32),
            pltpu.VMEM((_SUP,), jnp.int32),
            pltpu.VMEM((_SUP,), jnp.int32),
            pltpu.VMEM((_SUP, _H), jnp.float32),
            pltpu.VMEM((_SUP, _D), jnp.float32),
            pltpu.VMEM((_SUB, _HF), jnp.float32),
            pltpu.VMEM((_SUB, _HF), jnp.float32),
            pltpu.VMEM((_SUP, _OUT), jnp.float32),
            pltpu.VMEM_SHARED((_N, _OUT), jnp.float32),
            pltpu.SemaphoreType.DMA,
            pltpu.SemaphoreType.DMA,
            pltpu.SemaphoreType.DMA,
            pltpu.SemaphoreType.DMA,
            pltpu.SemaphoreType.DMA,
            pltpu.SemaphoreType.DMA,
            pltpu.SemaphoreType.DMA,
            pltpu.SemaphoreType.DMA,
        ],
    )
    return f(src, dst, w, dd, h)


def _tc_final_body(a1_ref, a2_ref, b1_ref, b2_ref, xin_ref, xout_ref):
    v1 = (a1_ref[0] + a1_ref[1]) * (1.0 / _H) + b1_ref[...]
    xin_ref[...] = jnp.where(v1 > 0, v1, jnp.exp(v1) - 1.0)
    v2 = (a2_ref[0] + a2_ref[1]) * (1.0 / _H) + b2_ref[...]
    xout_ref[...] = jnp.where(v2 > 0, v2, jnp.exp(v2) - 1.0)


def _tc_final(acc1, acc2, b1, b2):
    nb = _N // _BM
    return pl.pallas_call(
        _tc_final_body,
        grid=(nb,),
        in_specs=[
            pl.BlockSpec((_NC, _BM, _OUT), lambda i: (0, i, 0)),
            pl.BlockSpec((_NC, _BM, _OUT), lambda i: (0, i, 0)),
            pl.BlockSpec((1, _OUT), lambda i: (0, 0)),
            pl.BlockSpec((1, _OUT), lambda i: (0, 0)),
        ],
        out_specs=[
            pl.BlockSpec((_BM, _OUT), lambda i: (i, 0)),
            pl.BlockSpec((_BM, _OUT), lambda i: (i, 0)),
        ],
        out_shape=[
            jax.ShapeDtypeStruct((_N, _OUT), jnp.float32),
            jax.ShapeDtypeStruct((_N, _OUT), jnp.float32),
        ],
    )(acc1, acc2, b1, b2)


def _blockdiag(a):
    eye = jnp.eye(_H, dtype=jnp.float32)
    return (a[0][:, :, None] * eye[:, None, :]).reshape(_HF, _H)


def kernel(x, edge_index, W1, a_s1, a_d1, b1, W2, a_s2, a_d2, b2,
           lw1, lb1, lw2, lb2):
    e0 = edge_index[0]
    e1 = edge_index[1]
    p1 = jnp.concatenate([_blockdiag(a_s1), _blockdiag(a_d1)], axis=1)
    p2 = jnp.concatenate([_blockdiag(a_s2), _blockdiag(a_d2)], axis=1)
    h1, h2, tbl, x_self = _tc_dense(
        x, W1.T, W2.T, p1, p2, lw1.T, lb1[None, :], lw2.T, lb2[None, :])
    w1, w2, d1p = _sc_edgea(e0, e1, tbl)
    d2p = _sc_edgeb(e0, w2)
    dd1, dd2 = _tc_comb(d1p, d2p)
    acc1 = _sc_agg(e0, e1, w1, dd1, h1)
    acc2 = _sc_agg(e1, e0, w2, dd2, h2)
    x_in, x_out = _tc_final(acc1, acc2, b1[None, :], b2[None, :])
    return (x_in, x_out, x_self)


# speedup vs baseline: 1.0276x; 1.0276x over previous
import functools

import jax
import jax.numpy as jnp
from jax import lax
from jax.experimental import pallas as pl
from jax.experimental.pallas import tpu as pltpu
from jax.experimental.pallas import tpu_sc as plsc

_N = 10000
_E = 320000
_D = 128
_H = 16
_OUT = 128
_HF = _H * _OUT

_NC = 2
_NS = 16
_NW = _NC * _NS
_EPW = _E // _NW

_BM = 400
_G1 = 40
_GB = 80
_G2 = 16


def _tc_dense_body(x_ref, w1t, w2t, p1, p2, lw1t, lb1, lw2t, lb2,
                   h1o, h2o, to, xso):
    xb = x_ref[...]
    h1 = jnp.dot(xb, w1t[...], preferred_element_type=jnp.float32)
    h1o[...] = h1
    a1 = jnp.dot(h1, p1[...], preferred_element_type=jnp.float32)
    h2 = jnp.dot(xb, w2t[...], preferred_element_type=jnp.float32)
    h2o[...] = h2
    a2 = jnp.dot(h2, p2[...], preferred_element_type=jnp.float32)
    to[...] = jnp.concatenate(
        [a1[:, :_H], a2[:, _H:], a1[:, _H:], a2[:, :_H],
         jnp.zeros((_BM, 64), jnp.float32)], axis=1)
    x1 = jnp.dot(xb, lw1t[...], preferred_element_type=jnp.float32) + lb1[...]
    x1 = jnp.where(x1 > 0, x1, jnp.exp(x1) - 1.0)
    xs = jnp.dot(x1, lw2t[...], preferred_element_type=jnp.float32) + lb2[...]
    xso[...] = jnp.where(xs > 0, xs, jnp.exp(xs) - 1.0)


def _tc_dense(x, w1t, w2t, p1, p2, lw1t, lb1, lw2t, lb2):
    nb = _N // _BM
    full = lambda shape: pl.BlockSpec(shape, lambda i: (0, 0))
    return pl.pallas_call(
        _tc_dense_body,
        grid=(nb,),
        in_specs=[
            pl.BlockSpec((_BM, _D), lambda i: (i, 0)),
            full((_D, _HF)), full((_D, _HF)),
            full((_HF, 2 * _H)), full((_HF, 2 * _H)),
            full((_D, 4 * _OUT)), full((1, 4 * _OUT)),
            full((4 * _OUT, _OUT)), full((1, _OUT)),
        ],
        out_specs=[
            pl.BlockSpec((_BM, _HF), lambda i: (i, 0)),
            pl.BlockSpec((_BM, _HF), lambda i: (i, 0)),
            pl.BlockSpec((_BM, _D), lambda i: (i, 0)),
            pl.BlockSpec((_BM, _OUT), lambda i: (i, 0)),
        ],
        out_shape=[
            jax.ShapeDtypeStruct((_N, _HF), jnp.float32),
            jax.ShapeDtypeStruct((_N, _HF), jnp.float32),
            jax.ShapeDtypeStruct((_N, _D), jnp.float32),
            jax.ShapeDtypeStruct((_N, _OUT), jnp.float32),
        ],
    )(x, w1t, w2t, p1, p2, lw1t, lb1, lw2t, lb2)


def _zero_rows128(buf, nrows_buf):
    def zrow(j, _):
        for t in range(8):
            buf[j, pl.ds(t * 16, 16)] = jnp.zeros((16,), jnp.float32)
        return 0
    lax.fori_loop(0, nrows_buf, zrow, 0)


def _zero_table128(buf, dst, s, nrows_buf):
    @pl.when(s < 10)
    def _():
        for t in range(1000 // nrows_buf):
            pltpu.sync_copy(buf, dst.at[pl.ds(s * 1000 + t * nrows_buf,
                                              nrows_buf)])
        rem = 1000 % nrows_buf
        if rem:
            pltpu.sync_copy(
                buf.at[pl.ds(0, rem)],
                dst.at[pl.ds(s * 1000 + 1000 - rem, rem)])


def _sc_edgea_body(e0h, e1h, th,
                   w1h, w2h, d1ph,
                   idx0, idx1, bta, btb, bw1, bw2, bw1p, d1s,
                   sem0, sem1):
    c = lax.axis_index("c")
    s = lax.axis_index("s")
    wid = s * _NC + c

    _zero_rows128(bw1p, _G1)
    _zero_table128(bw1p, d1s, s, _G1)
    plsc.subcore_barrier()

    def chunk(k, _):
        b = pl.multiple_of(wid * _EPW + k * _G1, 8)
        cpi = pltpu.async_copy(e0h.at[pl.ds(b, _G1)], idx0, sem0)
        cpj = pltpu.async_copy(e1h.at[pl.ds(b, _G1)], idx1, sem1)
        cpi.wait()
        cp0 = pltpu.async_copy(th.at[idx0], bta, sem0)
        cpj.wait()
        cp1 = pltpu.async_copy(th.at[idx1], btb, sem1)
        cp0.wait()
        cp1.wait()

        def edge(i, _):
            s1 = bta[i, pl.ds(0, _H)] + btb[i, pl.ds(2 * _H, _H)]
            s2 = btb[i, pl.ds(3 * _H, _H)] + bta[i, pl.ds(_H, _H)]
            w1 = jnp.exp(jnp.maximum(s1, 0.2 * s1))
            bw1[i] = w1
            bw1p[i, pl.ds(0, _H)] = w1
            bw2[i] = jnp.exp(jnp.maximum(s2, 0.2 * s2))
            return 0
        lax.fori_loop(0, _G1, edge, 0)

        pltpu.sync_copy(bw1, w1h.at[pl.ds(b, _G1)])
        pltpu.sync_copy(bw2, w2h.at[pl.ds(b, _G1)])
        pltpu.sync_copy(bw1p, d1s.at[idx1], add=True)
        return 0
    lax.fori_loop(0, _EPW // _G1, chunk, 0)

    plsc.subcore_barrier()

    @pl.when(s < 10)
    def _():
        rows = pl.ds(s * 1000, 1000)
        pltpu.sync_copy(d1s.at[rows], d1ph.at[c].at[rows])


def _sc_edgea(e0, e1, tbl):
    mesh = plsc.VectorSubcoreMesh(core_axis_name="c", subcore_axis_name="s")
    f = pl.kernel(
        _sc_edgea_body,
        out_type=(
            jax.ShapeDtypeStruct((_E, _H), jnp.float32),
            jax.ShapeDtypeStruct((_E, _H), jnp.float32),
            jax.ShapeDtypeStruct((_NC, _N, _D), jnp.float32),
        ),
        mesh=mesh,
        scratch_types=[
            pltpu.VMEM((_G1,), jnp.int32),
            pltpu.VMEM((_G1,), jnp.int32),
            pltpu.VMEM((_G1, _D), jnp.float32),
            pltpu.VMEM((_G1, _D), jnp.float32),
            pltpu.VMEM((_G1, _H), jnp.float32),
            pltpu.VMEM((_G1, _H), jnp.float32),
            pltpu.VMEM((_G1, _D), jnp.float32),
            pltpu.VMEM_SHARED((_N, _D), jnp.float32),
            pltpu.SemaphoreType.DMA,
            pltpu.SemaphoreType.DMA,
        ],
    )
    return f(e0, e1, tbl)


def _sc_edgeb_body(e0h, w2h, d2ph,
                   idx0, bw2, bw2p, d2s):
    c = lax.axis_index("c")
    s = lax.axis_index("s")
    wid = s * _NC + c

    _zero_rows128(bw2p, _GB)
    _zero_table128(bw2p, d2s, s, _GB)
    plsc.subcore_barrier()

    def chunk(k, _):
        b = pl.multiple_of(wid * _EPW + k * _GB, 8)
        pltpu.sync_copy(e0h.at[pl.ds(b, _GB)], idx0)
        pltpu.sync_copy(w2h.at[pl.ds(b, _GB)], bw2)

        def edge(i, _):
            bw2p[i, pl.ds(0, _H)] = bw2[i]
            return 0
        lax.fori_loop(0, _GB, edge, 0)
        pltpu.sync_copy(bw2p, d2s.at[idx0], add=True)
        return 0
    lax.fori_loop(0, _EPW // _GB, chunk, 0)

    plsc.subcore_barrier()

    @pl.when(s < 10)
    def _():
        rows = pl.ds(s * 1000, 1000)
        pltpu.sync_copy(d2s.at[rows], d2ph.at[c].at[rows])


def _sc_edgeb(e0, w2):
    mesh = plsc.VectorSubcoreMesh(core_axis_name="c", subcore_axis_name="s")
    f = pl.kernel(
        _sc_edgeb_body,
        out_type=jax.ShapeDtypeStruct((_NC, _N, _D), jnp.float32),
        mesh=mesh,
        scratch_types=[
            pltpu.VMEM((_GB,), jnp.int32),
            pltpu.VMEM((_GB, _H), jnp.float32),
            pltpu.VMEM((_GB, _D), jnp.float32),
            pltpu.VMEM_SHARED((_N, _D), jnp.float32),
        ],
    )
    return f(e0, w2)


def _tc_comb_body(d1p_ref, d2p_ref, dd1_ref, dd2_ref):
    z = jnp.zeros((_BM, _D - _H), jnp.float32)
    dd1_ref[...] = jnp.concatenate(
        [1.0 / (d1p_ref[0, :, :_H] + d1p_ref[1, :, :_H] + 1e-16), z], axis=1)
    dd2_ref[...] = jnp.concatenate(
        [1.0 / (d2p_ref[0, :, :_H] + d2p_ref[1, :, :_H] + 1e-16), z], axis=1)


def _tc_comb(d1p, d2p):
    nb = _N // _BM
    return pl.pallas_call(
        _tc_comb_body,
        grid=(nb,),
        in_specs=[
            pl.BlockSpec((_NC, _BM, _D), lambda i: (0, i, 0)),
            pl.BlockSpec((_NC, _BM, _D), lambda i: (0, i, 0)),
        ],
        out_specs=[
            pl.BlockSpec((_BM, _D), lambda i: (i, 0)),
            pl.BlockSpec((_BM, _D), lambda i: (i, 0)),
        ],
        out_shape=[
            jax.ShapeDtypeStruct((_N, _D), jnp.float32),
            jax.ShapeDtypeStruct((_N, _D), jnp.float32),
        ],
    )(d1p, d2p)


_SUP = 40
_SUB = 8
_NSB = _SUP // _SUB


def _sc_agg_body(srch, dsth, wh, ddh, hh,
                 acco,
                 idxs0, idxd0, idxs1, idxd1, bw, bdg, bh0, bh1, bmsg, accs,
                 semd, semh0, semh1, semi0, semj0, semi1, semj1, semw):
    c = lax.axis_index("c")
    s = lax.axis_index("s")
    wid = s * _NC + c
    base = wid * _EPW
    bhs = (bh0, bh1)
    semhs = (semh0, semh1)
    idxss = (idxs0, idxs1)
    idxds = (idxd0, idxd1)
    semis = (semi0, semi1)
    semjs = (semj0, semj1)

    for r in range(_SUP):
        for t in range(8):
            bmsg[r, pl.ds(t * 16, 16)] = jnp.zeros((16,), jnp.float32)

    @pl.when(s < 10)
    def _():
        def zcp(t, _):
            pltpu.sync_copy(bmsg, accs.at[pl.ds(s * 1000 + t * _SUP, _SUP)])
            return 0
        lax.fori_loop(0, 1000 // _SUP, zcp, 0)
    plsc.subcore_barrier()

    def _super(k, p):
        b = pl.multiple_of(base + k * _SUP, 8)
        idxs = idxss[p]
        idxd = idxds[p]
        cpi = pltpu.async_copy(srch.at[pl.ds(b, _SUP)], idxs, semis[p])
        cpj = pltpu.async_copy(dsth.at[pl.ds(b, _SUP)], idxd, semjs[p])
        cpw = pltpu.async_copy(wh.at[pl.ds(b, _SUP)], bw, semw)
        cpj.wait()
        cpd = pltpu.async_copy(ddh.at[idxd], bdg, semd)
        cpi.wait()
        cps = [None] * _NSB
        cps[0] = pltpu.async_copy(hh.at[idxs.at[pl.ds(0, _SUB)]], bh0, semh0)
        cpw.wait()
        cpd.wait()

        for t in range(_NSB):
            if t + 1 < _NSB:
                cps[t + 1] = pltpu.async_copy(
                    hh.at[idxs.at[pl.ds((t + 1) * _SUB, _SUB)]],
                    bhs[(t + 1) % 2], semhs[(t + 1) % 2])
            cps[t].wait()
            bh = bhs[t % 2]

            def edge(i, _):
                e = t * _SUB + i
                al = bw[e] * bdg[e, pl.ds(0, _H)]
                mv = [jnp.zeros((16,), jnp.float32) for _ in range(8)]
                for hd in range(_H):
                    av = jnp.full((16,), al[hd], jnp.float32)
                    for j in range(8):
                        mv[j] = mv[j] + av * bh[i, pl.ds(hd * _OUT + j * 16,
                                                         16)]
                for j in range(8):
                    bmsg[e, pl.ds(j * 16, 16)] = mv[j]
                return 0
            lax.fori_loop(0, _SUB, edge, 0)

        pltpu.sync_copy(bmsg, accs.at[idxd], add=True)

    def chunk(k, _):
        _super(k, 0)
        return 0
    lax.fori_loop(0, _EPW // _SUP, chunk, 0)

    plsc.subcore_barrier()

    @pl.when(s < 10)
    def _():
        rows = pl.ds(s * 1000, 1000)
        pltpu.sync_copy(accs.at[rows], acco.at[c].at[rows])


def _sc_agg(src, dst, w, dd, h):
    mesh = plsc.VectorSubcoreMesh(core_axis_name="c", subcore_axis_name="s")
    f = pl.kernel(
        _sc_agg_body,
        out_type=jax.ShapeDtypeStruct((_NC, _N, _OUT), jnp.float32),
        mesh=mesh,
        scratch_types=[
            pltpu.VMEM((_SUP,), jnp.int32),
            pltpu.VMEM((_SUP,), jnp.int32),
            pltpu.VMEM((_SUP,), jnp.int32),
            pltpu.VMEM((_SUP,), jnp.int32),
            pltpu.VMEM((_SUP, _H), jnp.float32),
            pltpu.VMEM((_SUP, _D), jnp.float32),
            pltpu.VMEM((_SUB, _HF), jnp.float32),
            pltpu.VMEM((_SUB, _HF), jnp.float32),
            pltpu.VMEM((_SUP, _OUT), jnp.float32),
            pltpu.VMEM_SHARED((_N, _OUT), jnp.float32),
            pltpu.SemaphoreType.DMA,
            pltpu.SemaphoreType.DMA,
            pltpu.SemaphoreType.DMA,
            pltpu.SemaphoreType.DMA,
            pltpu.SemaphoreType.DMA,
            pltpu.SemaphoreType.DMA,
            pltpu.SemaphoreType.DMA,
            pltpu.SemaphoreType.DMA,
        ],
    )
    return f(src, dst, w, dd, h)


def _tc_final_body(a1_ref, a2_ref, b1_ref, b2_ref, xin_ref, xout_ref):
    v1 = (a1_ref[0] + a1_ref[1]) * (1.0 / _H) + b1_ref[...]
    xin_ref[...] = jnp.where(v1 > 0, v1, jnp.exp(v1) - 1.0)
    v2 = (a2_ref[0] + a2_ref[1]) * (1.0 / _H) + b2_ref[...]
    xout_ref[...] = jnp.where(v2 > 0, v2, jnp.exp(v2) - 1.0)


def _tc_final(acc1, acc2, b1, b2):
    nb = _N // _BM
    return pl.pallas_call(
        _tc_final_body,
        grid=(nb,),
        in_specs=[
            pl.BlockSpec((_NC, _BM, _OUT), lambda i: (0, i, 0)),
            pl.BlockSpec((_NC, _BM, _OUT), lambda i: (0, i, 0)),
            pl.BlockSpec((1, _OUT), lambda i: (0, 0)),
            pl.BlockSpec((1, _OUT), lambda i: (0, 0)),
        ],
        out_specs=[
            pl.BlockSpec((_BM, _OUT), lambda i: (i, 0)),
            pl.BlockSpec((_BM, _OUT), lambda i: (i, 0)),
        ],
        out_shape=[
            jax.ShapeDtypeStruct((_N, _OUT), jnp.float32),
            jax.ShapeDtypeStruct((_N, _OUT), jnp.float32),
        ],
    )(acc1, acc2, b1, b2)


def _blockdiag(a):
    eye = jnp.eye(_H, dtype=jnp.float32)
    return (a[0][:, :, None] * eye[:, None, :]).reshape(_HF, _H)


def kernel(x, edge_index, W1, a_s1, a_d1, b1, W2, a_s2, a_d2, b2,
           lw1, lb1, lw2, lb2):
    e0 = edge_index[0]
    e1 = edge_index[1]
    p1 = jnp.concatenate([_blockdiag(a_s1), _blockdiag(a_d1)], axis=1)
    p2 = jnp.concatenate([_blockdiag(a_s2), _blockdiag(a_d2)], axis=1)
    h1, h2, tbl, x_self = _tc_dense(
        x, W1.T, W2.T, p1, p2, lw1.T, lb1[None, :], lw2.T, lb2[None, :])
    w1, w2, d1p = _sc_edgea(e0, e1, tbl)
    d2p = _sc_edgeb(e0, w2)
    dd1, dd2 = _tc_comb(d1p, d2p)
    acc1 = _sc_agg(e0, e1, w1, dd1, h1)
    acc2 = _sc_agg(e1, e0, w2, dd2, h2)
    x_in, x_out = _tc_final(acc1, acc2, b1[None, :], b2[None, :])
    return (x_in, x_out, x_self)
